# speedup vs baseline: 6.1375x; 6.1375x over previous
import functools

import jax
import jax.numpy as jnp
from jax.experimental import pallas as pl
from jax.experimental.pallas import tpu as pltpu

_L = 512
_C = 128
_OUT = 128
_K = 10


def _body(x_ref, w0_ref, w1_ref, w2_ref, bc_ref, wln_ref, wlc_ref, bl_ref,
          out_ref):
    xb = x_ref[0]
    hi = jax.lax.Precision.HIGHEST
    dg = functools.partial(jax.lax.dot_general, precision=hi,
                           preferred_element_type=jnp.float32)

    G = dg(xb, xb, (((0,), (0,)), ((), ())))
    d2 = jnp.sum(xb * xb, axis=0)
    rs = jax.lax.rsqrt(d2)
    dif = d2[None, :] + d2[:, None] - 2.0 * G
    cosw = G * rs[:, None] * rs[None, :]

    zcol = jnp.zeros((_C, 1), jnp.float32)
    xl = jnp.concatenate([xb[:, 1:], zcol], axis=1)
    xr = jnp.concatenate([zcol, xb[:, :-1]], axis=1)
    cT = (dg(xr, w0_ref[...], (((0,), (1,)), ((), ())))
          + dg(xb, w1_ref[...], (((0,), (1,)), ((), ())))
          + dg(xl, w2_ref[...], (((0,), (1,)), ((), ())))
          + bc_ref[...])

    P_n = dg(xb, wln_ref[...], (((0,), (1,)), ((), ())))
    P_cb = dg(xb, wlc_ref[...], (((0,), (1,)), ((), ()))) + bl_ref[...]

    iota = jax.lax.broadcasted_iota(jnp.int32, (_L, _L), 1)
    g = jnp.full((_L, _OUT), -jnp.inf, jnp.float32)
    for _ in range(_K):
        m = jnp.min(dif, axis=1)
        is_min = dif == m[:, None]
        idxv = jnp.min(jnp.where(is_min, iota, _L), axis=1)
        onehot = (iota == idxv[:, None]).astype(jnp.float32)
        Pg = dg(onehot, P_n, (((1,), (0,)), ((), ())))
        w = jnp.sum(onehot * cosw, axis=1)
        g = jnp.maximum(g, (Pg + P_cb) * w[:, None])
        dif = jnp.where(onehot > 0.0, jnp.inf, dif)

    act = jnp.maximum(cT + g, 0.0)
    pooled = jnp.max(act.reshape(_L // 2, 2, _OUT), axis=1)
    out_ref[0] = pooled


def kernel(x, num_frms, Wc, bc, Wl, bl):
    del num_frms
    bs = x.shape[0]
    w0 = Wc[:, :, 0]
    w1 = Wc[:, :, 1]
    w2 = Wc[:, :, 2]
    wln = Wl[:, :_C]
    wlc = Wl[:, _C:]
    full = lambda s: pl.BlockSpec(s, lambda b: (0,) * len(s))
    out = pl.pallas_call(
        _body,
        grid=(bs,),
        in_specs=[
            pl.BlockSpec((1, _C, _L), lambda b: (b, 0, 0)),
            full((_OUT, _C)), full((_OUT, _C)), full((_OUT, _C)),
            full((1, _OUT)),
            full((_OUT, _C)), full((_OUT, _C)),
            full((1, _OUT)),
        ],
        out_specs=pl.BlockSpec((1, _L // 2, _OUT), lambda b: (b, 0, 0)),
        out_shape=jax.ShapeDtypeStruct((bs, _L // 2, _OUT), jnp.float32),
    )(x, w0, w1, w2, bc.reshape(1, _OUT), wln, wlc, bl.reshape(1, _OUT))
    return jnp.transpose(out, (0, 2, 1))
